# baseline (speedup 1.0000x reference)
import functools

import jax
import jax.numpy as jnp
from jax import lax
from jax.experimental import pallas as pl
from jax.experimental.pallas import tpu as pltpu
from jax.experimental.pallas import tpu_sc as plsc

EMBED_DIM = 64
BATCH = 4096
T = 30
NF = 6
OUT_D = 320
NDB = OUT_D // 8
NC, NS = 2, 16
NW = NC * NS
BPW = BATCH // NW
PAD = 136


def _rsqrt16(s):
    i = lax.bitcast_convert_type(s, jnp.int32)
    y = lax.bitcast_convert_type(jnp.int32(0x5F3759DF) - (i >> 1), jnp.float32)
    for _ in range(3):
        y = y * (jnp.float32(1.5) - jnp.float32(0.5) * s * y * y)
    return y


def _body(xT, table, out,
          i0, i1, i2, f0, f1, f2, r0_, r1_, r2_, s0, s1, s2,
          acc, p0, p1, p2, p3,
          sI0, sI1, sI2, sF0, sF1, sF2, sR0, sR1, sR2, sS0, sS1, sS2,
          sPr0, sPr1, sPr2, sPr3, sPw0, sPw1, sPw2, sPw3):
    wid = lax.axis_index("s") * NC + lax.axis_index("c")
    I = (i0, i1, i2)
    F = (f0, f1, f2)
    R = (r0_, r1_, r2_)
    S = (s0, s1, s2)
    P = (p0, p1, p2, p3)
    sI = (sI0, sI1, sI2)
    sF = (sF0, sF1, sF2)
    sR = (sR0, sR1, sR2)
    sS = (sS0, sS1, sS2)
    sPr = (sPr0, sPr1, sPr2, sPr3)
    sPw = (sPw0, sPw1, sPw2, sPw3)

    lane = lax.iota(jnp.int32, 16)

    def fire_idx(t, k):
        pltpu.async_copy(
            xT.at[t, :, pl.ds(wid * BPW, BPW)], I[k], sI[k])

    def wait_idx(k):
        pltpu.make_async_copy(
            xT.at[0, :, pl.ds(0, BPW)], I[k], sI[k]).wait()

    def fire_f5(k):
        pltpu.async_copy(table.at[I[k].at[5]], F[k], sF[k])

    def wait_f5(k):
        pltpu.make_async_copy(table.at[I[k].at[5]], F[k], sF[k]).wait()

    def fire_ri(tk, i, k):
        pltpu.async_copy(table.at[I[tk].at[i]], R[k], sR[k])

    def wait_ri(k):
        pltpu.make_async_copy(table.at[I[k % 3].at[0]], R[k], sR[k]).wait()

    def fire_stage(t, i, k):
        for db in range(8):
            pltpu.async_copy(
                S[k].at[pl.ds(db * 1024, 1024)],
                out.at[t, i * 8 + db, wid],
                sS[k],
            )

    def wait_stage(k):
        for db in range(8):
            pltpu.make_async_copy(
                S[k].at[pl.ds(db * 1024, 1024)],
                out.at[0, 0, 0],
                sS[k],
            ).wait()

    def z_body(d, c):
        acc[pl.ds(d * 16, 16)] = jnp.zeros((16,), jnp.float32)
        return c
    lax.fori_loop(0, OUT_D * PAD // 16, z_body, 0)

    pltpu.sync_copy(xT.at[0, :, pl.ds(wid * BPW, BPW)], I[0])
    fire_f5(0)
    fire_ri(0, 0, 0)
    fire_ri(0, 1, 1)

    def compute_ti(t, dt, i):
        s3 = (2 * dt + i) % 3

        if i == 0:
            @pl.when(t + 1 < T)
            def _():
                fire_idx(t + 1, (dt + 1) % 3)
            wait_f5(dt)
        if i <= 2:
            fire_ri(dt, i + 2, (2 * dt + i + 2) % 3)
        elif i == 3:
            @pl.when(t + 1 < T)
            def _():
                wait_idx((dt + 1) % 3)
                fire_f5((dt + 1) % 3)
        else:
            @pl.when(t + 1 < T)
            def _():
                fire_ri((dt + 1) % 3, 0, (2 * dt + 2) % 3)
                fire_ri((dt + 1) % 3, 1, (2 * dt) % 3)

        wait_ri(s3)
        sidx = 2 * dt + i
        @pl.when(t * 5 + i >= 3)
        def _():
            wait_stage(sidx % 3)

        rows = R[s3]
        f5 = F[dt]
        st = S[sidx % 3]

        def b_body(b, c):
            bv = jnp.full((16,), 0, jnp.int32) + b
            for jj in range(4):
                vi = rows[b, pl.ds(jj * 16, 16)]
                v5 = f5[b, pl.ds(jj * 16, 16)]
                v = vi + v5
                sidx_f = (lane + jj * 16) * 128 + bv
                plsc.store_scatter(st, [sidx_f], v)
                aidx_f = (lane + (i * 64 + jj * 16)) * PAD + bv
                plsc.addupdate_scatter(acc, [aidx_f], v * v)
            return c
        lax.fori_loop(0, BPW, b_body, 0)
        fire_stage(t, i, sidx % 3)

    def step_body(stp, c):
        for dt in range(3):
            t = stp * 3 + dt
            for i in range(5):
                compute_ti(t, dt, i)
        return c
    lax.fori_loop(0, T // 3, step_body, 0)
    wait_stage(0)
    wait_stage(1)
    wait_stage(2)

    def sc_body(d, c):
        for bb in range(8):
            sl = pl.ds(d * PAD + bb * 16, 16)
            a = acc[sl]
            y = _rsqrt16(a)
            acc[sl] = jnp.where(
                a >= jnp.float32(1e-24), y, jnp.float32(1e12))
        return c
    lax.fori_loop(0, OUT_D, sc_body, 0)

    NCH = T * (NDB // 2)

    def fire_rd(c, k):
        t = c // (NDB // 2)
        g = c % (NDB // 2)
        pltpu.async_copy(
            out.at[t, pl.ds(g * 2, 2), wid], P[k], sPr[k])

    def wait_rd(k):
        pltpu.make_async_copy(
            out.at[0, pl.ds(0, 2), 0], P[k], sPr[k]).wait()

    def fire_wr(c, k):
        t = c // (NDB // 2)
        g = c % (NDB // 2)
        pltpu.async_copy(
            P[k], out.at[t, pl.ds(g * 2, 2), wid], sPw[k])

    def wait_wr(k):
        pltpu.make_async_copy(
            P[k], out.at[0, pl.ds(0, 2), 0], sPw[k]).wait()

    fire_rd(0, 0)
    fire_rd(1, 1)

    def p2_chunk(c, k):
        wait_rd(k)
        g2 = (c % (NDB // 2)) * 2
        pp = P[k]
        for dl in range(2):
            for ds_ in range(8):
                d = (g2 + dl) * 8 + ds_
                for bb in range(8):
                    psl = pl.ds(ds_ * 128 + bb * 16, 16)
                    asl = pl.ds(d * PAD + bb * 16, 16)
                    pp[dl, psl] = pp[dl, psl] * acc[asl]
        fire_wr(c, k)

        @pl.when(c + 2 < NCH)
        def _():
            @pl.when(c >= 2)
            def _():
                wait_wr((k + 2) % 4)
            fire_rd(c + 2, (k + 2) % 4)

    def p2_step(stp, c):
        for kk in range(4):
            ch = stp * 4 + kk
            p2_chunk(ch, kk)
        return c
    lax.fori_loop(0, NCH // 4, p2_step, 0)
    wait_wr(0)
    wait_wr(1)
    wait_wr(2)
    wait_wr(3)


_sc_call = functools.partial(
    pl.kernel,
    out_type=jax.ShapeDtypeStruct((T, NDB, NW, 1024), jnp.float32),
    mesh=plsc.VectorSubcoreMesh(core_axis_name="c", subcore_axis_name="s"),
    compiler_params=pltpu.CompilerParams(
        use_tc_tiling_on_sc=False, needs_layout_passes=False),
    scratch_types=(
        [pltpu.VMEM((NF, BPW), jnp.int32)] * 3
        + [pltpu.VMEM((BPW, EMBED_DIM), jnp.float32)] * 3
        + [pltpu.VMEM((BPW, EMBED_DIM), jnp.float32)] * 3
        + [pltpu.VMEM((8192,), jnp.float32)] * 3
        + [pltpu.VMEM((OUT_D * PAD,), jnp.float32)]
        + [pltpu.VMEM((2, 1024), jnp.float32)] * 4
        + [pltpu.SemaphoreType.DMA] * 20
    ),
)(_body)


def kernel(x, table):
    xT = x.transpose(1, 2, 0)
    out5 = _sc_call(xT, table).reshape(T, NDB, NW, 8, 128)
    return out5.transpose(2, 4, 0, 1, 3).reshape(BATCH, T, OUT_D)


# speedup vs baseline: 1.8138x; 1.8138x over previous
import functools

import jax
import jax.numpy as jnp
from jax import lax
from jax.experimental import pallas as pl
from jax.experimental.pallas import tpu as pltpu
from jax.experimental.pallas import tpu_sc as plsc

EMBED_DIM = 64
BATCH = 4096
T = 30
NF = 6
OUT_D = (NF - 1) * EMBED_DIM

NC = 2
NS = 16
NW = NC * NS
HALF = BATCH // 2
PAIRS_PER_W = HALF // (2 * NW)
IDX_PER_PAIR = 2 * T * NF


def _rsqrt16(s):
    i = lax.bitcast_convert_type(s, jnp.int32)
    y = lax.bitcast_convert_type(jnp.int32(0x5F3759DF) - (i >> 1), jnp.float32)
    for _ in range(3):
        y = y * (jnp.float32(1.5) - jnp.float32(0.5) * s * y * y)
    return y


def _body(x_hbm, table_hbm, out_hbm,
          i0, i1, i2, r0_, r1_, r2_, o0, o1, o2,
          si0, si1, si2, sg0, sg1, sg2, so0, so1, so2):
    wid = lax.axis_index("s") * NC + lax.axis_index("c")
    base_p = wid * PAIRS_PER_W
    ibuf = (i0, i1, i2)
    rbuf = (r0_, r1_, r2_)
    obuf = (o0, o1, o2)
    sem_i = (si0, si1, si2)
    sem_g = (sg0, sg1, sg2)
    sem_o = (so0, so1, so2)

    def fire_idx(p, j):
        pltpu.async_copy(x_hbm.at[pl.ds(base_p + p, 1)], ibuf[j], sem_i[j])

    def wait_idx(j):
        pltpu.make_async_copy(
            x_hbm.at[pl.ds(0, 1)], ibuf[j], sem_i[j]).wait()

    def fire_gather(j):
        pltpu.async_copy(table_hbm.at[ibuf[j].at[0]], rbuf[j], sem_g[j])

    def wait_gather(j):
        pltpu.make_async_copy(
            table_hbm.at[ibuf[j].at[0]], rbuf[j], sem_g[j]).wait()

    def fire_out(p, j):
        pltpu.async_copy(
            obuf[j], out_hbm.at[pl.ds((base_p + p) * 2, 2)], sem_o[j])

    def wait_out(j):
        pltpu.make_async_copy(
            obuf[j], out_hbm.at[pl.ds(0, 2)], sem_o[j]).wait()

    def compute(j):
        rows_v = rbuf[j]
        out_v = obuf[j]
        for be in range(2):
            def t_body(t, acc):
                base = (be * T + t) * NF
                f5 = [rows_v[base + 5, pl.ds(jj * 16, 16)] for jj in range(4)]
                new_acc = list(acc)
                for i in range(5):
                    for jj in range(4):
                        v = rows_v[base + i, pl.ds(jj * 16, 16)] + f5[jj]
                        out_v[be, t, pl.ds(i * 64 + jj * 16, 16)] = v
                        k = i * 4 + jj
                        new_acc[k] = acc[k] + v * v
                return tuple(new_acc)

            zero = jnp.zeros((16,), jnp.float32)
            acc = lax.fori_loop(0, T, t_body, tuple(zero for _ in range(20)))

            scales = []
            for k in range(20):
                s = acc[k]
                y = _rsqrt16(s)
                scales.append(
                    jnp.where(s >= jnp.float32(1e-24), y, jnp.float32(1e12))
                )

            def scale_body(t, carry2):
                for i in range(5):
                    for jj in range(4):
                        sl = pl.ds(i * 64 + jj * 16, 16)
                        out_v[be, t, sl] = out_v[be, t, sl] * scales[i * 4 + jj]
                return carry2

            lax.fori_loop(0, T, scale_body, 0)

    pltpu.sync_copy(x_hbm.at[pl.ds(base_p + 0, 1)], ibuf[0])
    pltpu.sync_copy(x_hbm.at[pl.ds(base_p + 1, 1)], ibuf[1])
    fire_gather(0)
    fire_gather(1)
    fire_idx(2, 2)

    def step_body(s, carry):
        for j in range(3):
            p = s * 3 + j

            @pl.when(p < PAIRS_PER_W)
            def _():
                wait_gather(j)

                @pl.when(p + 3 < PAIRS_PER_W)
                def _():
                    fire_idx(p + 3, j)

            @pl.when(p + 2 < PAIRS_PER_W)
            def _():
                wait_idx((j + 2) % 3)
                fire_gather((j + 2) % 3)

            @pl.when(p < PAIRS_PER_W)
            def _():
                @pl.when(p >= 3)
                def _():
                    wait_out(j)

                compute(j)
                fire_out(p, j)
        return carry

    lax.fori_loop(0, 11, step_body, 0)
    wait_out(2)
    wait_out(0)
    wait_out(1)


_sc_call = functools.partial(
    pl.kernel,
    out_type=jax.ShapeDtypeStruct((HALF, T, OUT_D), jnp.float32),
    mesh=plsc.VectorSubcoreMesh(core_axis_name="c", subcore_axis_name="s"),
    compiler_params=pltpu.CompilerParams(use_tc_tiling_on_sc=False),
    scratch_types=(
        [pltpu.VMEM((1, IDX_PER_PAIR), jnp.int32)] * 3
        + [pltpu.VMEM((IDX_PER_PAIR, EMBED_DIM), jnp.float32)] * 3
        + [pltpu.VMEM((2, T, OUT_D), jnp.float32)] * 3
        + [pltpu.SemaphoreType.DMA] * 9
    ),
)(_body)


def kernel(x, table):
    x2 = x.reshape(BATCH * T * NF // IDX_PER_PAIR, IDX_PER_PAIR)
    n = x2.shape[0] // 2
    o1 = _sc_call(x2[:n], table)
    o2 = _sc_call(x2[n:], table)
    return jnp.concatenate([o1, o2], axis=0)


# speedup vs baseline: 1.9319x; 1.0651x over previous
import functools

import jax
import jax.numpy as jnp
from jax import lax
from jax.experimental import pallas as pl
from jax.experimental.pallas import tpu as pltpu
from jax.experimental.pallas import tpu_sc as plsc

EMBED_DIM = 64
BATCH = 4096
T = 30
NF = 6
OUT_D = (NF - 1) * EMBED_DIM

NC = 2
NS = 16
NW = NC * NS
PAIRS_PER_W = BATCH // (2 * NW)
IDX_PER_PAIR = 2 * T * NF


def _rsqrt16(s):
    i = lax.bitcast_convert_type(s, jnp.int32)
    y = lax.bitcast_convert_type(jnp.int32(0x5F3759DF) - (i >> 1), jnp.float32)
    for _ in range(3):
        y = y * (jnp.float32(1.5) - jnp.float32(0.5) * s * y * y)
    return y


def _body(x_hbm, table_hbm, out_hbm,
          i0, i1, i2, r0_, r1_, r2_, o0, o1, o2,
          si0, si1, si2, sg0, sg1, sg2, so0, so1, so2):
    wid = lax.axis_index("s") * NC + lax.axis_index("c")
    base_p = wid * PAIRS_PER_W
    ibuf = (i0, i1, i2)
    rbuf = (r0_, r1_, r2_)
    obuf = (o0, o1, o2)
    sem_i = (si0, si1, si2)
    sem_g = (sg0, sg1, sg2)
    sem_o = (so0, so1, so2)

    def fire_idx(p, j):
        pltpu.async_copy(x_hbm.at[pl.ds(base_p + p, 1)], ibuf[j], sem_i[j])

    def wait_idx(j):
        pltpu.make_async_copy(
            x_hbm.at[pl.ds(0, 1)], ibuf[j], sem_i[j]).wait()

    def fire_gather(j):
        pltpu.async_copy(table_hbm.at[ibuf[j].at[0]], rbuf[j], sem_g[j])

    def wait_gather(j):
        pltpu.make_async_copy(
            table_hbm.at[ibuf[j].at[0]], rbuf[j], sem_g[j]).wait()

    def fire_out(p, j):
        pltpu.async_copy(
            obuf[j], out_hbm.at[pl.ds((base_p + p) * 2, 2)], sem_o[j])

    def wait_out(j):
        pltpu.make_async_copy(
            obuf[j], out_hbm.at[pl.ds(0, 2)], sem_o[j]).wait()

    def compute(j):
        rows_v = rbuf[j]
        out_v = obuf[j]
        for be in range(2):
            def t_body(t, acc):
                base = (be * T + t) * NF
                f5 = [rows_v[base + 5, pl.ds(jj * 16, 16)] for jj in range(4)]
                new_acc = list(acc)
                for i in range(5):
                    for jj in range(4):
                        v = rows_v[base + i, pl.ds(jj * 16, 16)] + f5[jj]
                        out_v[be, t, pl.ds(i * 64 + jj * 16, 16)] = v
                        k = i * 4 + jj
                        new_acc[k] = acc[k] + v * v
                return tuple(new_acc)

            zero = jnp.zeros((16,), jnp.float32)
            acc = lax.fori_loop(0, T, t_body, tuple(zero for _ in range(20)))

            scales = []
            for k in range(20):
                s = acc[k]
                y = _rsqrt16(s)
                scales.append(
                    jnp.where(s >= jnp.float32(1e-24), y, jnp.float32(1e12))
                )

            def scale_body(t, carry2):
                for i in range(5):
                    for jj in range(4):
                        sl = pl.ds(i * 64 + jj * 16, 16)
                        out_v[be, t, sl] = out_v[be, t, sl] * scales[i * 4 + jj]
                return carry2

            lax.fori_loop(0, T, scale_body, 0)

    pltpu.sync_copy(x_hbm.at[pl.ds(base_p + 0, 1)], ibuf[0])
    pltpu.sync_copy(x_hbm.at[pl.ds(base_p + 1, 1)], ibuf[1])
    fire_gather(0)
    fire_gather(1)
    fire_idx(2, 2)

    def step_body(s, carry):
        for j in range(3):
            p = s * 3 + j

            @pl.when(p < PAIRS_PER_W)
            def _():
                wait_gather(j)

                @pl.when(p + 3 < PAIRS_PER_W)
                def _():
                    fire_idx(p + 3, j)

            @pl.when(p + 2 < PAIRS_PER_W)
            def _():
                wait_idx((j + 2) % 3)
                fire_gather((j + 2) % 3)

            @pl.when(p < PAIRS_PER_W)
            def _():
                @pl.when(p >= 3)
                def _():
                    wait_out(j)

                compute(j)
                fire_out(p, j)
        return carry

    lax.fori_loop(0, 22, step_body, 0)
    wait_out(1)
    wait_out(2)
    wait_out(0)


_sc_call = functools.partial(
    pl.kernel,
    out_type=jax.ShapeDtypeStruct((BATCH, T, OUT_D), jnp.float32),
    mesh=plsc.VectorSubcoreMesh(core_axis_name="c", subcore_axis_name="s"),
    compiler_params=pltpu.CompilerParams(use_tc_tiling_on_sc=False),
    scratch_types=(
        [pltpu.VMEM((1, IDX_PER_PAIR), jnp.int32)] * 3
        + [pltpu.VMEM((IDX_PER_PAIR, EMBED_DIM), jnp.float32)] * 3
        + [pltpu.VMEM((2, T, OUT_D), jnp.float32)] * 3
        + [pltpu.SemaphoreType.DMA] * 9
    ),
)(_body)


def kernel(x, table):
    x2 = x.reshape(BATCH * T * NF // IDX_PER_PAIR, IDX_PER_PAIR)
    return _sc_call(x2, table)
